# baseline (speedup 1.0000x reference)
import functools

import numpy as np

import jax
import jax.numpy as jnp
from jax.experimental import pallas as pl
from jax.experimental.pallas import tpu as pltpu

_N = 512
_G = _N // 4
_EO = 30

_pm_np = np.zeros((_N, _N), np.float32)
_r = np.arange(_N)
_pm_np[_r, 4 * (_r % _G) + _r // _G] = 1.0
_PM = _pm_np
_PMT = _pm_np.T.copy()
_esel_np = np.zeros((4 * 4 * _EO, _EO), np.float32)
for _k in range(4):
    _esel_np[120 * _k + 30 * _k:120 * _k + 30 * (_k + 1), :] = np.eye(_EO)
_ESEL = _esel_np
_FOLD = np.kron(np.ones((4, 1), np.float32), np.eye(_EO, dtype=np.float32))
_RM = np.kron(np.eye(4, dtype=np.float32), np.ones((1, _EO), np.float32))
_MASK32 = np.kron(np.eye(4, dtype=np.float32), np.ones((1, 32), np.float32))
_MASKBD = np.kron(np.eye(4, dtype=np.float32), np.ones((32, _EO), np.float32))


def _body(x_ref, adj_ref, adj4_ref, gcn_w_ref, ew1_ref, b1_ref, ew2_ref,
          b2_ref, nw_ref, nb_ref, wih_ref, bih_ref, bhh_ref, fow_ref,
          fob_ref, pm_ref, pmt_ref, rm_ref, m32_ref, mbd_ref, esel_ref,
          fold_ref,
          out_ref,
          adjt_s, w1s_s, w1t4_s, b1t4_s, rv_s, w2b_s, b2t_s, nwt_s, wg_s,
          wx_s, fot_s, h_s, c4b_s, a4_s, acc_s, subp_s,
          *, gb, n, nt):
    it = pl.program_id(1)

    @pl.when(it == 0)
    def _weight_prep():
        adjt_s[...] = jnp.swapaxes(adj4_ref[...], 1, 2)
        w1s_s[...] = ew1_ref[:, 0:128].T
        w1t = ew1_ref[:, 128:256].T
        w1t4_s[...] = jnp.concatenate([w1t] * 4, axis=1)
        b1row = b1_ref[...]
        b1t4_s[...] = jnp.concatenate([b1row] * 4, axis=1)
        vrow = ew1_ref[:, 256:257].T
        vt = jnp.concatenate([vrow] * 4, axis=1)
        rv_s[...] = m32_ref[...] * vt
        w2t = ew2_ref[...].T
        w2c = jnp.concatenate([w2t] * 4, axis=1)
        w2tile = jnp.concatenate([w2c] * 4, axis=0)
        w2b_s[...] = mbd_ref[...] * w2tile
        b2row = b2_ref[...]
        b2t_s[...] = jnp.concatenate([b2row] * 4, axis=1)
        nwt_s[...] = nw_ref[...].T
        wg_s[...] = wih_ref[:, 0:13].T
        wx_s[...] = wih_ref[:, 13:141].T
        fot_s[...] = fow_ref[...].T

    @pl.when(it == 0)
    def _prep():
        support = jnp.dot(x_ref[0], gcn_w_ref[...],
                          preferred_element_type=jnp.float32)
        h = jax.nn.relu(jnp.dot(adj_ref[...], support,
                                preferred_element_type=jnp.float32))
        h_s[...] = h
        c4b_s[...] = (jnp.dot(h, w1t4_s[...],
                              preferred_element_type=jnp.float32)
                      + b1t4_s[...])
        hp = jnp.dot(pm_ref[...], h, preferred_element_type=jnp.float32)
        w1s = w1s_s[...]
        a4_s[...] = jnp.concatenate(
            [jnp.dot(hp[128 * k:128 * (k + 1)], w1s,
                     preferred_element_type=jnp.float32)
             for k in range(4)], axis=1)

    flat4 = adjt_s[pl.ds(it * gb, gb), :, :].reshape(gb * n, 4)
    term = jnp.dot(flat4, rv_s[...], preferred_element_type=jnp.float32)
    x1 = (term.reshape(gb, n, 128)
          + a4_s[pl.ds(it * gb, gb), :][:, None, :]
          + c4b_s[...][None, :, :])
    s1 = jax.nn.sigmoid(x1).reshape(gb * n, 128)
    o2 = (jnp.dot(s1, w2b_s[...], preferred_element_type=jnp.float32)
          + b2t_s[...])
    s2 = jax.nn.sigmoid(o2)
    mf = jnp.where(flat4 != 0.0, 1.0, 0.0)
    mm = jnp.dot(mf, rm_ref[...], preferred_element_type=jnp.float32)
    m2 = (s2 * mm).reshape(gb, n, 120)
    subp_s[pl.ds(it * gb, gb), :] = jnp.sum(m2, axis=1)
    colsum = jnp.sum(m2, axis=0)

    @pl.when(it == 0)
    def _init():
        acc_s[...] = colsum

    @pl.when(it != 0)
    def _accum():
        acc_s[...] = acc_s[...] + colsum

    @pl.when(it == nt - 1)
    def _head():
        addf = jnp.dot(acc_s[...], fold_ref[...],
                       preferred_element_type=jnp.float32)
        subp = subp_s[...]
        stack = jnp.concatenate(
            [jnp.dot(subp, esel_ref[120 * k:120 * (k + 1), :],
                     preferred_element_type=jnp.float32)
             for k in range(4)], axis=0)
        subn = jnp.dot(pmt_ref[...], stack,
                       preferred_element_type=jnp.float32)
        agg = addf - subn
        xg = jax.nn.sigmoid(jnp.dot(agg, nwt_s[...],
                                    preferred_element_type=jnp.float32)
                            + nb_ref[...])
        h = h_s[...]
        gi = (jnp.dot(xg, wg_s[...], preferred_element_type=jnp.float32)
              + jnp.dot(h, wx_s[...], preferred_element_type=jnp.float32)
              + bih_ref[...])
        bhh = bhh_ref[...]
        r = jax.nn.sigmoid(gi[:, 0:64] + bhh[:, 0:64])
        z = jax.nn.sigmoid(gi[:, 64:128] + bhh[:, 64:128])
        nng = jnp.tanh(gi[:, 128:192] + r * bhh[:, 128:192])
        hn = (1.0 - z) * nng
        out_ref[0] = (jnp.dot(hn, fot_s[...],
                              preferred_element_type=jnp.float32)
                      + fob_ref[...])


def kernel(x, node_adj, gcn_w, e_w1, e_b1, e_w2, e_b2, n_w, n_b,
           w_ih, w_hh, b_ih, b_hh, fo_w, fo_b):
    B, N, NF = x.shape
    EH = e_w1.shape[0]
    EO = e_w2.shape[0]
    GO = n_w.shape[0]
    HID = w_hh.shape[1]
    GRU_IN = w_ih.shape[1]
    G = N // 4
    GB = 32
    NT = G // GB
    f32 = jnp.float32

    adj4 = node_adj.reshape(G, 4, N)
    pm = jnp.asarray(_PM)
    pmt = jnp.asarray(_PMT)
    rm = jnp.asarray(_RM)
    m32 = jnp.asarray(_MASK32)
    mbd = jnp.asarray(_MASKBD)
    esel = jnp.asarray(_ESEL)
    fold = jnp.asarray(_FOLD)

    full = lambda shape: pl.BlockSpec(shape, lambda b, it: (0,) * len(shape))
    out = pl.pallas_call(
        functools.partial(_body, gb=GB, n=N, nt=NT),
        grid=(B, NT),
        in_specs=[
            pl.BlockSpec((1, N, NF), lambda b, it: (b, 0, 0)),
            full((N, N)),
            full((G, 4, N)),
            full((NF, NF)),
            full((EH, 2 * NF + 1)),
            full((1, EH)),
            full((EO, EH)),
            full((1, EO)),
            full((GO, EO)),
            full((1, GO)),
            full((3 * HID, GRU_IN)),
            full((1, 3 * HID)),
            full((1, 3 * HID)),
            full((1, HID)),
            full((1, 1)),
            full((N, N)),
            full((N, N)),
            full((4, 4 * EO)),
            full((4, 4 * EH)),
            full((4 * EH, 4 * EO)),
            full((4 * 4 * EO, EO)),
            full((4 * EO, EO)),
        ],
        out_specs=pl.BlockSpec((1, N, 1), lambda b, it: (b, 0, 0)),
        out_shape=jax.ShapeDtypeStruct((B, N, 1), jnp.float32),
        scratch_shapes=[
            pltpu.VMEM((G, N, 4), f32),
            pltpu.VMEM((NF, EH), f32),
            pltpu.VMEM((NF, 4 * EH), f32),
            pltpu.VMEM((1, 4 * EH), f32),
            pltpu.VMEM((4, 4 * EH), f32),
            pltpu.VMEM((4 * EH, 4 * EO), f32),
            pltpu.VMEM((1, 4 * EO), f32),
            pltpu.VMEM((EO, GO), f32),
            pltpu.VMEM((GO, 3 * HID), f32),
            pltpu.VMEM((NF, 3 * HID), f32),
            pltpu.VMEM((HID, 1), f32),
            pltpu.VMEM((N, NF), f32),
            pltpu.VMEM((N, 4 * EH), f32),
            pltpu.VMEM((G, 4 * EH), f32),
            pltpu.VMEM((N, 4 * EO), f32),
            pltpu.VMEM((G, 4 * EO), f32),
        ],
        compiler_params=pltpu.CompilerParams(
            dimension_semantics=("parallel", "arbitrary")),
    )(x, node_adj, adj4, gcn_w, e_w1, e_b1.reshape(1, EH),
      e_w2, e_b2.reshape(1, EO), n_w, n_b.reshape(1, GO), w_ih,
      b_ih.reshape(1, 3 * HID), b_hh.reshape(1, 3 * HID),
      fo_w.reshape(1, HID), fo_b.reshape(1, 1), pm, pmt, rm, m32, mbd,
      esel, fold)

    return out[:, None, :, :]


# speedup vs baseline: 1.6147x; 1.6147x over previous
import functools

import numpy as np

import jax
import jax.numpy as jnp
from jax.experimental import pallas as pl
from jax.experimental.pallas import tpu as pltpu

_N = 512
_G = _N // 4
_EO = 30

_pm_np = np.zeros((_N, _N), np.float32)
_r = np.arange(_N)
_pm_np[_r, 4 * (_r % _G) + _r // _G] = 1.0
_PM = _pm_np
_PMT = _pm_np.T.copy()
_esel_np = np.zeros((4 * 4 * _EO, _EO), np.float32)
for _k in range(4):
    _esel_np[120 * _k + 30 * _k:120 * _k + 30 * (_k + 1), :] = np.eye(_EO)
_ESEL = _esel_np
_FOLD = np.kron(np.ones((4, 1), np.float32), np.eye(_EO, dtype=np.float32))
_RM = np.kron(np.eye(4, dtype=np.float32), np.ones((1, _EO), np.float32))
_MASK32 = np.kron(np.eye(4, dtype=np.float32), np.ones((1, 32), np.float32))
_MASKBD = np.kron(np.eye(4, dtype=np.float32), np.ones((32, _EO), np.float32))


def _body(x_ref, adj_ref, adj4_ref, gcn_w_ref, ew1_ref, b1_ref, ew2_ref,
          b2_ref, nw_ref, nb_ref, wih_ref, bih_ref, bhh_ref, fow_ref,
          fob_ref, pm_ref, pmt_ref, rm_ref, m32_ref, mbd_ref, esel_ref,
          fold_ref,
          out_ref,
          adjt_s, w1s_s, w1t4_s, b1t4_s, rv_s, w2b_s, b2t_s, nwt_s, wg_s,
          wx_s, fot_s, h_s, c4b_s, a4_s, acc_s, subp_s,
          *, gb, n, nt):
    b = pl.program_id(0)
    it = pl.program_id(1)

    @pl.when(jnp.logical_and(b == 0, it == 0))
    def _weight_prep():
        adjt_s[...] = jnp.swapaxes(adj4_ref[...], 1, 2)
        w1s_s[...] = ew1_ref[:, 0:128].T
        w1t = ew1_ref[:, 128:256].T
        w1t4_s[...] = jnp.concatenate([w1t] * 4, axis=1)
        b1row = b1_ref[...]
        b1t4_s[...] = jnp.concatenate([b1row] * 4, axis=1)
        vrow = ew1_ref[:, 256:257].T
        vt = jnp.concatenate([vrow] * 4, axis=1)
        rv_s[...] = 0.5 * (m32_ref[...] * vt)
        w2t = ew2_ref[...].T
        w2c = jnp.concatenate([w2t] * 4, axis=1)
        w2tile = jnp.concatenate([w2c] * 4, axis=0)
        w2full = mbd_ref[...] * w2tile
        w2b_s[...] = 0.25 * w2full
        b2row = b2_ref[...]
        b2t = jnp.concatenate([b2row] * 4, axis=1)
        b2t_s[...] = 0.5 * b2t + 0.25 * jnp.sum(w2full, axis=0,
                                                keepdims=True)
        nwt_s[...] = nw_ref[...].T
        wg_s[...] = wih_ref[:, 0:13].T
        wx_s[...] = wih_ref[:, 13:141].T
        fot_s[...] = fow_ref[...].T

    @pl.when(it == 0)
    def _prep():
        support = jnp.dot(x_ref[0], gcn_w_ref[...],
                          preferred_element_type=jnp.float32)
        h = jax.nn.relu(jnp.dot(adj_ref[...], support,
                                preferred_element_type=jnp.float32))
        h_s[...] = h
        c4b_s[...] = 0.5 * (jnp.dot(h, w1t4_s[...],
                                    preferred_element_type=jnp.float32)
                            + b1t4_s[...])
        hp = jnp.dot(pm_ref[...], h, preferred_element_type=jnp.float32)
        w1s = w1s_s[...]
        a4_s[...] = 0.5 * jnp.concatenate(
            [jnp.dot(hp[128 * k:128 * (k + 1)], w1s,
                     preferred_element_type=jnp.float32)
             for k in range(4)], axis=1)

    flat4 = adjt_s[pl.ds(it * gb, gb), :, :].reshape(gb * n, 4)
    term = jnp.dot(flat4, rv_s[...], preferred_element_type=jnp.float32)
    x1 = (term.reshape(gb, n, 128)
          + a4_s[pl.ds(it * gb, gb), :][:, None, :]
          + c4b_s[...][None, :, :])
    t1 = jnp.tanh(x1).reshape(gb * n, 128)
    o2h = (jnp.dot(t1, w2b_s[...], preferred_element_type=jnp.float32)
           + b2t_s[...])
    s2 = 0.5 * jnp.tanh(o2h) + 0.5
    mf = jnp.where(flat4 != 0.0, 1.0, 0.0)
    mm = jnp.dot(mf, rm_ref[...], preferred_element_type=jnp.float32)
    m2 = (s2 * mm).reshape(gb, n, 120)
    subp_s[pl.ds(it * gb, gb), :] = jnp.sum(m2, axis=1)
    colsum = jnp.sum(m2, axis=0)

    @pl.when(it == 0)
    def _init():
        acc_s[...] = colsum

    @pl.when(it != 0)
    def _accum():
        acc_s[...] = acc_s[...] + colsum

    @pl.when(it == nt - 1)
    def _head():
        addf = jnp.dot(acc_s[...], fold_ref[...],
                       preferred_element_type=jnp.float32)
        subp = subp_s[...]
        stack = jnp.concatenate(
            [jnp.dot(subp, esel_ref[120 * k:120 * (k + 1), :],
                     preferred_element_type=jnp.float32)
             for k in range(4)], axis=0)
        subn = jnp.dot(pmt_ref[...], stack,
                       preferred_element_type=jnp.float32)
        agg = addf - subn
        xg = jax.nn.sigmoid(jnp.dot(agg, nwt_s[...],
                                    preferred_element_type=jnp.float32)
                            + nb_ref[...])
        h = h_s[...]
        gi = (jnp.dot(xg, wg_s[...], preferred_element_type=jnp.float32)
              + jnp.dot(h, wx_s[...], preferred_element_type=jnp.float32)
              + bih_ref[...])
        bhh = bhh_ref[...]
        r = jax.nn.sigmoid(gi[:, 0:64] + bhh[:, 0:64])
        z = jax.nn.sigmoid(gi[:, 64:128] + bhh[:, 64:128])
        nng = jnp.tanh(gi[:, 128:192] + r * bhh[:, 128:192])
        hn = (1.0 - z) * nng
        out_ref[0] = (jnp.dot(hn, fot_s[...],
                              preferred_element_type=jnp.float32)
                      + fob_ref[...])


def kernel(x, node_adj, gcn_w, e_w1, e_b1, e_w2, e_b2, n_w, n_b,
           w_ih, w_hh, b_ih, b_hh, fo_w, fo_b):
    B, N, NF = x.shape
    EH = e_w1.shape[0]
    EO = e_w2.shape[0]
    GO = n_w.shape[0]
    HID = w_hh.shape[1]
    GRU_IN = w_ih.shape[1]
    G = N // 4
    GB = 32
    NT = G // GB
    f32 = jnp.float32

    adj4 = node_adj.reshape(G, 4, N)
    pm = jnp.asarray(_PM)
    pmt = jnp.asarray(_PMT)
    rm = jnp.asarray(_RM)
    m32 = jnp.asarray(_MASK32)
    mbd = jnp.asarray(_MASKBD)
    esel = jnp.asarray(_ESEL)
    fold = jnp.asarray(_FOLD)

    full = lambda shape: pl.BlockSpec(shape, lambda b, it: (0,) * len(shape))
    out = pl.pallas_call(
        functools.partial(_body, gb=GB, n=N, nt=NT),
        grid=(B, NT),
        in_specs=[
            pl.BlockSpec((1, N, NF), lambda b, it: (b, 0, 0)),
            full((N, N)),
            full((G, 4, N)),
            full((NF, NF)),
            full((EH, 2 * NF + 1)),
            full((1, EH)),
            full((EO, EH)),
            full((1, EO)),
            full((GO, EO)),
            full((1, GO)),
            full((3 * HID, GRU_IN)),
            full((1, 3 * HID)),
            full((1, 3 * HID)),
            full((1, HID)),
            full((1, 1)),
            full((N, N)),
            full((N, N)),
            full((4, 4 * EO)),
            full((4, 4 * EH)),
            full((4 * EH, 4 * EO)),
            full((4 * 4 * EO, EO)),
            full((4 * EO, EO)),
        ],
        out_specs=pl.BlockSpec((1, N, 1), lambda b, it: (b, 0, 0)),
        out_shape=jax.ShapeDtypeStruct((B, N, 1), jnp.float32),
        scratch_shapes=[
            pltpu.VMEM((G, N, 4), f32),
            pltpu.VMEM((NF, EH), f32),
            pltpu.VMEM((NF, 4 * EH), f32),
            pltpu.VMEM((1, 4 * EH), f32),
            pltpu.VMEM((4, 4 * EH), f32),
            pltpu.VMEM((4 * EH, 4 * EO), f32),
            pltpu.VMEM((1, 4 * EO), f32),
            pltpu.VMEM((EO, GO), f32),
            pltpu.VMEM((GO, 3 * HID), f32),
            pltpu.VMEM((NF, 3 * HID), f32),
            pltpu.VMEM((HID, 1), f32),
            pltpu.VMEM((N, NF), f32),
            pltpu.VMEM((N, 4 * EH), f32),
            pltpu.VMEM((G, 4 * EH), f32),
            pltpu.VMEM((N, 4 * EO), f32),
            pltpu.VMEM((G, 4 * EO), f32),
        ],
        compiler_params=pltpu.CompilerParams(
            dimension_semantics=("arbitrary", "arbitrary")),
    )(x, node_adj, adj4, gcn_w, e_w1, e_b1.reshape(1, EH),
      e_w2, e_b2.reshape(1, EO), n_w, n_b.reshape(1, GO), w_ih,
      b_ih.reshape(1, 3 * HID), b_hh.reshape(1, 3 * HID),
      fo_w.reshape(1, HID), fo_b.reshape(1, 1), pm, pmt, rm, m32, mbd,
      esel, fold)

    return out[:, None, :, :]


# speedup vs baseline: 1.6834x; 1.0425x over previous
import functools

import numpy as np

import jax
import jax.numpy as jnp
from jax.experimental import pallas as pl
from jax.experimental.pallas import tpu as pltpu

_N = 512
_G = _N // 4
_EO = 30

_pm_np = np.zeros((_N, _N), np.float32)
_r = np.arange(_N)
_pm_np[_r, 4 * (_r % _G) + _r // _G] = 1.0
_PM = _pm_np
_PMT = _pm_np.T.copy()
_esel_np = np.zeros((4 * 4 * _EO, _EO), np.float32)
for _k in range(4):
    _esel_np[120 * _k + 30 * _k:120 * _k + 30 * (_k + 1), :] = np.eye(_EO)
_ESEL = _esel_np
_FOLD = np.kron(np.ones((4, 1), np.float32), np.eye(_EO, dtype=np.float32))
_RM = np.kron(np.eye(4, dtype=np.float32), np.ones((1, _EO), np.float32))
_MASK32 = np.kron(np.eye(4, dtype=np.float32), np.ones((1, 32), np.float32))
_MASKBD = np.kron(np.eye(4, dtype=np.float32), np.ones((32, _EO), np.float32))


def _body(x_ref, adj_ref, adj4_ref, gcn_w_ref, ew1_ref, b1_ref, ew2_ref,
          b2_ref, nw_ref, nb_ref, wih_ref, bih_ref, bhh_ref, fow_ref,
          fob_ref, pm_ref, pmt_ref, rm_ref, m32_ref, mbd_ref, esel_ref,
          fold_ref,
          out_ref,
          w1s_s, w1t4_s, b1t4_s, rv_s, w2b_s, b2t_s, nwt_s, wg_s,
          wx_s, fot_s, h_s, c4b_s, a4_s, acc_s, subp_s,
          *, gb, n, nt):
    b = pl.program_id(0)
    it = pl.program_id(1)

    @pl.when(jnp.logical_and(b == 0, it == 0))
    def _weight_prep():
        w1s_s[...] = ew1_ref[:, 0:128].T
        w1t = ew1_ref[:, 128:256].T
        w1t4_s[...] = jnp.concatenate([w1t] * 4, axis=1)
        b1row = b1_ref[...]
        b1t4_s[...] = jnp.concatenate([b1row] * 4, axis=1)
        vrow = ew1_ref[:, 256:257].T
        vt = jnp.concatenate([vrow] * 4, axis=1)
        rv_s[...] = 0.5 * (m32_ref[...] * vt)
        w2t = ew2_ref[...].T
        w2c = jnp.concatenate([w2t] * 4, axis=1)
        w2tile = jnp.concatenate([w2c] * 4, axis=0)
        w2full = mbd_ref[...] * w2tile
        w2b_s[...] = 0.25 * w2full
        b2row = b2_ref[...]
        b2t = jnp.concatenate([b2row] * 4, axis=1)
        b2t_s[...] = 0.5 * b2t + 0.25 * jnp.sum(w2full, axis=0,
                                                keepdims=True)
        nwt_s[...] = nw_ref[...].T
        wg_s[...] = wih_ref[:, 0:13].T
        wx_s[...] = wih_ref[:, 13:141].T
        fot_s[...] = fow_ref[...].T

    @pl.when(it == 0)
    def _prep():
        support = jnp.dot(x_ref[0], gcn_w_ref[...],
                          preferred_element_type=jnp.float32)
        h = jax.nn.relu(jnp.dot(adj_ref[...], support,
                                preferred_element_type=jnp.float32))
        h_s[...] = h
        c4b_s[...] = 0.5 * (jnp.dot(h, w1t4_s[...],
                                    preferred_element_type=jnp.float32)
                            + b1t4_s[...])
        hp = jnp.dot(pm_ref[...], h, preferred_element_type=jnp.float32)
        w1s = w1s_s[...]
        a4_s[...] = 0.5 * jnp.concatenate(
            [jnp.dot(hp[128 * k:128 * (k + 1)], w1s,
                     preferred_element_type=jnp.float32)
             for k in range(4)], axis=1)

    adjslab = adj4_ref[...]
    rvb = jnp.broadcast_to(rv_s[...][None], (gb, 4, 128))
    term3 = jax.lax.dot_general(
        adjslab, rvb, (((1,), (1,)), ((0,), (0,))),
        preferred_element_type=jnp.float32)
    x1 = (term3
          + a4_s[pl.ds(it * gb, gb), :][:, None, :]
          + c4b_s[...][None, :, :])
    t1 = jnp.tanh(x1).reshape(gb * n, 128)
    o2h = (jnp.dot(t1, w2b_s[...], preferred_element_type=jnp.float32)
           + b2t_s[...])
    s2 = 0.5 * jnp.tanh(o2h) + 0.5
    mf4 = jnp.where(adjslab != 0.0, 1.0, 0.0)
    rmb = jnp.broadcast_to(rm_ref[...][None], (gb, 4, 120))
    mm3 = jax.lax.dot_general(
        mf4, rmb, (((1,), (1,)), ((0,), (0,))),
        preferred_element_type=jnp.float32)
    m2 = s2.reshape(gb, n, 120) * mm3
    subp_s[pl.ds(it * gb, gb), :] = jnp.sum(m2, axis=1)
    colsum = jnp.sum(m2, axis=0)

    @pl.when(it == 0)
    def _init():
        acc_s[...] = colsum

    @pl.when(it != 0)
    def _accum():
        acc_s[...] = acc_s[...] + colsum

    @pl.when(it == nt - 1)
    def _head():
        addf = jnp.dot(acc_s[...], fold_ref[...],
                       preferred_element_type=jnp.float32)
        subp = subp_s[...]
        stack = jnp.concatenate(
            [jnp.dot(subp, esel_ref[120 * k:120 * (k + 1), :],
                     preferred_element_type=jnp.float32)
             for k in range(4)], axis=0)
        subn = jnp.dot(pmt_ref[...], stack,
                       preferred_element_type=jnp.float32)
        agg = addf - subn
        xg = jax.nn.sigmoid(jnp.dot(agg, nwt_s[...],
                                    preferred_element_type=jnp.float32)
                            + nb_ref[...])
        h = h_s[...]
        gi = (jnp.dot(xg, wg_s[...], preferred_element_type=jnp.float32)
              + jnp.dot(h, wx_s[...], preferred_element_type=jnp.float32)
              + bih_ref[...])
        bhh = bhh_ref[...]
        r = jax.nn.sigmoid(gi[:, 0:64] + bhh[:, 0:64])
        z = jax.nn.sigmoid(gi[:, 64:128] + bhh[:, 64:128])
        nng = jnp.tanh(gi[:, 128:192] + r * bhh[:, 128:192])
        hn = (1.0 - z) * nng
        out_ref[0] = (jnp.dot(hn, fot_s[...],
                              preferred_element_type=jnp.float32)
                      + fob_ref[...])


def kernel(x, node_adj, gcn_w, e_w1, e_b1, e_w2, e_b2, n_w, n_b,
           w_ih, w_hh, b_ih, b_hh, fo_w, fo_b):
    B, N, NF = x.shape
    EH = e_w1.shape[0]
    EO = e_w2.shape[0]
    GO = n_w.shape[0]
    HID = w_hh.shape[1]
    GRU_IN = w_ih.shape[1]
    G = N // 4
    GB = 32
    NT = G // GB
    f32 = jnp.float32

    adj4 = node_adj.reshape(G, 4, N)
    pm = jnp.asarray(_PM)
    pmt = jnp.asarray(_PMT)
    rm = jnp.asarray(_RM)
    m32 = jnp.asarray(_MASK32)
    mbd = jnp.asarray(_MASKBD)
    esel = jnp.asarray(_ESEL)
    fold = jnp.asarray(_FOLD)

    full = lambda shape: pl.BlockSpec(shape, lambda b, it: (0,) * len(shape))
    out = pl.pallas_call(
        functools.partial(_body, gb=GB, n=N, nt=NT),
        grid=(B, NT),
        in_specs=[
            pl.BlockSpec((1, N, NF), lambda b, it: (b, 0, 0)),
            full((N, N)),
            pl.BlockSpec((GB, 4, N), lambda b, it: (it, 0, 0)),
            full((NF, NF)),
            full((EH, 2 * NF + 1)),
            full((1, EH)),
            full((EO, EH)),
            full((1, EO)),
            full((GO, EO)),
            full((1, GO)),
            full((3 * HID, GRU_IN)),
            full((1, 3 * HID)),
            full((1, 3 * HID)),
            full((1, HID)),
            full((1, 1)),
            full((N, N)),
            full((N, N)),
            full((4, 4 * EO)),
            full((4, 4 * EH)),
            full((4 * EH, 4 * EO)),
            full((4 * 4 * EO, EO)),
            full((4 * EO, EO)),
        ],
        out_specs=pl.BlockSpec((1, N, 1), lambda b, it: (b, 0, 0)),
        out_shape=jax.ShapeDtypeStruct((B, N, 1), jnp.float32),
        scratch_shapes=[
            pltpu.VMEM((NF, EH), f32),
            pltpu.VMEM((NF, 4 * EH), f32),
            pltpu.VMEM((1, 4 * EH), f32),
            pltpu.VMEM((4, 4 * EH), f32),
            pltpu.VMEM((4 * EH, 4 * EO), f32),
            pltpu.VMEM((1, 4 * EO), f32),
            pltpu.VMEM((EO, GO), f32),
            pltpu.VMEM((GO, 3 * HID), f32),
            pltpu.VMEM((NF, 3 * HID), f32),
            pltpu.VMEM((HID, 1), f32),
            pltpu.VMEM((N, NF), f32),
            pltpu.VMEM((N, 4 * EH), f32),
            pltpu.VMEM((G, 4 * EH), f32),
            pltpu.VMEM((N, 4 * EO), f32),
            pltpu.VMEM((G, 4 * EO), f32),
        ],
        compiler_params=pltpu.CompilerParams(
            dimension_semantics=("arbitrary", "arbitrary")),
    )(x, node_adj, adj4, gcn_w, e_w1, e_b1.reshape(1, EH),
      e_w2, e_b2.reshape(1, EO), n_w, n_b.reshape(1, GO), w_ih,
      b_ih.reshape(1, 3 * HID), b_hh.reshape(1, 3 * HID),
      fo_w.reshape(1, HID), fo_b.reshape(1, 1), pm, pmt, rm, m32, mbd,
      esel, fold)

    return out[:, None, :, :]


# speedup vs baseline: 1.7267x; 1.0257x over previous
import functools

import numpy as np

import jax
import jax.numpy as jnp
from jax.experimental import pallas as pl
from jax.experimental.pallas import tpu as pltpu

_N = 512
_G = _N // 4
_EO = 30

_pm_np = np.zeros((_N, _N), np.float32)
_r = np.arange(_N)
_pm_np[_r, 4 * (_r % _G) + _r // _G] = 1.0
_PM = _pm_np
_PMT = _pm_np.T.copy()
_esel_np = np.zeros((4 * 4 * _EO, _EO), np.float32)
for _k in range(4):
    _esel_np[120 * _k + 30 * _k:120 * _k + 30 * (_k + 1), :] = np.eye(_EO)
_ESEL = _esel_np
_FOLD = np.kron(np.ones((4, 1), np.float32), np.eye(_EO, dtype=np.float32))
_RM = np.kron(np.eye(4, dtype=np.float32), np.ones((1, _EO), np.float32))
_MASK32 = np.kron(np.eye(4, dtype=np.float32), np.ones((1, 32), np.float32))
_MASKBD = np.kron(np.eye(4, dtype=np.float32), np.ones((32, _EO), np.float32))


def _body(x_ref, adj_ref, adj4_ref, gcn_w_ref, ew1_ref, b1_ref, ew2_ref,
          b2_ref, nw_ref, nb_ref, wih_ref, bih_ref, bhh_ref, fow_ref,
          fob_ref, pm_ref, pmt_ref, rm_ref, m32_ref, mbd_ref, esel_ref,
          fold_ref,
          out_ref,
          w1s_s, w1t4_s, b1t4_s, rv_s, w2b_s, b2t_s, nwt_s, wg_s,
          wx_s, fot_s, h_s, c4b_s, a4_s, acc_s, subp_s,
          *, gb, n, nt):
    b = pl.program_id(0)
    it = pl.program_id(1)

    @pl.when(jnp.logical_and(b == 0, it == 0))
    def _weight_prep():
        w1s_s[...] = ew1_ref[:, 0:128].T
        w1t = ew1_ref[:, 128:256].T
        w1t4_s[...] = jnp.concatenate([w1t] * 4, axis=1)
        b1row = b1_ref[...]
        b1t4_s[...] = jnp.concatenate([b1row] * 4, axis=1)
        vrow = ew1_ref[:, 256:257].T
        vt = jnp.concatenate([vrow] * 4, axis=1)
        rv_s[...] = 0.5 * (m32_ref[...] * vt)
        w2t = ew2_ref[...].T
        w2c = jnp.concatenate([w2t] * 4, axis=1)
        w2tile = jnp.concatenate([w2c] * 4, axis=0)
        w2full = mbd_ref[...] * w2tile
        w2b_s[...] = 0.25 * w2full
        b2row = b2_ref[...]
        b2t = jnp.concatenate([b2row] * 4, axis=1)
        b2t_s[...] = 0.5 * b2t + 0.25 * jnp.sum(w2full, axis=0,
                                                keepdims=True)
        nwt_s[...] = nw_ref[...].T
        wg_s[...] = wih_ref[:, 0:13].T
        wx_s[...] = wih_ref[:, 13:141].T
        fot_s[...] = fow_ref[...].T

    @pl.when(it == 0)
    def _prep():
        support = jnp.dot(x_ref[0], gcn_w_ref[...],
                          preferred_element_type=jnp.float32)
        h = jax.nn.relu(jnp.dot(adj_ref[...], support,
                                preferred_element_type=jnp.float32))
        h_s[...] = h
        c4b_s[...] = 0.5 * (jnp.dot(h, w1t4_s[...],
                                    preferred_element_type=jnp.float32)
                            + b1t4_s[...])
        hp = jnp.dot(pm_ref[...], h, preferred_element_type=jnp.float32)
        w1s = w1s_s[...]
        a4_s[...] = 0.5 * jnp.concatenate(
            [jnp.dot(hp[128 * k:128 * (k + 1)], w1s,
                     preferred_element_type=jnp.float32)
             for k in range(4)], axis=1)

    adjslab = adj4_ref[...]
    rvb = jnp.broadcast_to(rv_s[...][None], (gb, 4, 128))
    term3 = jax.lax.dot_general(
        adjslab, rvb, (((1,), (1,)), ((0,), (0,))),
        preferred_element_type=jnp.float32)
    x1 = (term3
          + a4_s[pl.ds(it * gb, gb), :][:, None, :]
          + c4b_s[...][None, :, :])
    t1 = jnp.tanh(x1).reshape(gb * n, 128)
    o2h = (jnp.dot(t1, w2b_s[...], preferred_element_type=jnp.float32)
           + b2t_s[...])
    s2 = 0.5 * jnp.tanh(o2h) + 0.5
    mf4 = jnp.where(adjslab != 0.0, 1.0, 0.0)
    rmb = jnp.broadcast_to(rm_ref[...][None], (gb, 4, 120))
    mm3 = jax.lax.dot_general(
        mf4, rmb, (((1,), (1,)), ((0,), (0,))),
        preferred_element_type=jnp.float32)
    m2 = s2.reshape(gb, n, 120) * mm3
    subp_s[pl.ds(it * gb, gb), :] = jnp.sum(m2, axis=1)
    colsum = jnp.sum(m2, axis=0)

    @pl.when(it == 0)
    def _init():
        acc_s[...] = colsum

    @pl.when(it != 0)
    def _accum():
        acc_s[...] = acc_s[...] + colsum

    @pl.when(it == nt - 1)
    def _head():
        addf = jnp.dot(acc_s[...], fold_ref[...],
                       preferred_element_type=jnp.float32)
        subp = subp_s[...]
        stack = jnp.concatenate(
            [jnp.dot(subp, esel_ref[120 * k:120 * (k + 1), :],
                     preferred_element_type=jnp.float32)
             for k in range(4)], axis=0)
        subn = jnp.dot(pmt_ref[...], stack,
                       preferred_element_type=jnp.float32)
        agg = addf - subn
        xg = jax.nn.sigmoid(jnp.dot(agg, nwt_s[...],
                                    preferred_element_type=jnp.float32)
                            + nb_ref[...])
        h = h_s[...]
        gi = (jnp.dot(xg, wg_s[...], preferred_element_type=jnp.float32)
              + jnp.dot(h, wx_s[...], preferred_element_type=jnp.float32)
              + bih_ref[...])
        bhh = bhh_ref[...]
        r = jax.nn.sigmoid(gi[:, 0:64] + bhh[:, 0:64])
        z = jax.nn.sigmoid(gi[:, 64:128] + bhh[:, 64:128])
        nng = jnp.tanh(gi[:, 128:192] + r * bhh[:, 128:192])
        hn = (1.0 - z) * nng
        out_ref[0] = (jnp.dot(hn, fot_s[...],
                              preferred_element_type=jnp.float32)
                      + fob_ref[...])


def kernel(x, node_adj, gcn_w, e_w1, e_b1, e_w2, e_b2, n_w, n_b,
           w_ih, w_hh, b_ih, b_hh, fo_w, fo_b):
    B, N, NF = x.shape
    EH = e_w1.shape[0]
    EO = e_w2.shape[0]
    GO = n_w.shape[0]
    HID = w_hh.shape[1]
    GRU_IN = w_ih.shape[1]
    G = N // 4
    GB = 64
    NT = G // GB
    f32 = jnp.float32

    adj4 = node_adj.reshape(G, 4, N)
    pm = jnp.asarray(_PM)
    pmt = jnp.asarray(_PMT)
    rm = jnp.asarray(_RM)
    m32 = jnp.asarray(_MASK32)
    mbd = jnp.asarray(_MASKBD)
    esel = jnp.asarray(_ESEL)
    fold = jnp.asarray(_FOLD)

    full = lambda shape: pl.BlockSpec(shape, lambda b, it: (0,) * len(shape))
    out = pl.pallas_call(
        functools.partial(_body, gb=GB, n=N, nt=NT),
        grid=(B, NT),
        in_specs=[
            pl.BlockSpec((1, N, NF), lambda b, it: (b, 0, 0)),
            full((N, N)),
            pl.BlockSpec((GB, 4, N), lambda b, it: (it, 0, 0)),
            full((NF, NF)),
            full((EH, 2 * NF + 1)),
            full((1, EH)),
            full((EO, EH)),
            full((1, EO)),
            full((GO, EO)),
            full((1, GO)),
            full((3 * HID, GRU_IN)),
            full((1, 3 * HID)),
            full((1, 3 * HID)),
            full((1, HID)),
            full((1, 1)),
            full((N, N)),
            full((N, N)),
            full((4, 4 * EO)),
            full((4, 4 * EH)),
            full((4 * EH, 4 * EO)),
            full((4 * 4 * EO, EO)),
            full((4 * EO, EO)),
        ],
        out_specs=pl.BlockSpec((1, N, 1), lambda b, it: (b, 0, 0)),
        out_shape=jax.ShapeDtypeStruct((B, N, 1), jnp.float32),
        scratch_shapes=[
            pltpu.VMEM((NF, EH), f32),
            pltpu.VMEM((NF, 4 * EH), f32),
            pltpu.VMEM((1, 4 * EH), f32),
            pltpu.VMEM((4, 4 * EH), f32),
            pltpu.VMEM((4 * EH, 4 * EO), f32),
            pltpu.VMEM((1, 4 * EO), f32),
            pltpu.VMEM((EO, GO), f32),
            pltpu.VMEM((GO, 3 * HID), f32),
            pltpu.VMEM((NF, 3 * HID), f32),
            pltpu.VMEM((HID, 1), f32),
            pltpu.VMEM((N, NF), f32),
            pltpu.VMEM((N, 4 * EH), f32),
            pltpu.VMEM((G, 4 * EH), f32),
            pltpu.VMEM((N, 4 * EO), f32),
            pltpu.VMEM((G, 4 * EO), f32),
        ],
        compiler_params=pltpu.CompilerParams(
            dimension_semantics=("arbitrary", "arbitrary")),
    )(x, node_adj, adj4, gcn_w, e_w1, e_b1.reshape(1, EH),
      e_w2, e_b2.reshape(1, EO), n_w, n_b.reshape(1, GO), w_ih,
      b_ih.reshape(1, 3 * HID), b_hh.reshape(1, 3 * HID),
      fo_w.reshape(1, HID), fo_b.reshape(1, 1), pm, pmt, rm, m32, mbd,
      esel, fold)

    return out[:, None, :, :]
